# baseline (speedup 1.0000x reference)
import jax
import jax.numpy as jnp
from jax import lax
from jax.experimental import pallas as pl
from jax.experimental.pallas import tpu as pltpu

_ROWS = 16 * 2048
_C = 256
_CH_ROWS = 2048
_NCHUNK = _ROWS // _CH_ROWS
_NBUF = 4


def _kl_kernel(pmu, pls_, fmu, fls, out_ref, bufs, sems):
    srcs = (pmu, pls_, fmu, fls)

    def dma(b, ci, i):
        r0 = ci * _CH_ROWS
        return pltpu.make_async_copy(
            srcs[i].at[pl.ds(r0, _CH_ROWS), :], bufs.at[b, i], sems.at[b, i])

    def issue(b, ci):
        for i in range(4):
            dma(b, ci, i).start()

    def wait(b, ci):
        for i in range(4):
            dma(b, ci, i).wait()

    for b in range(_NBUF):
        issue(b, b)

    def group_body(g, acc):
        for b in range(_NBUF):
            ci = g * _NBUF + b
            wait(b, ci)
            pls = bufs[b, 1]
            fls = bufs[b, 3]
            d = bufs[b, 2] - bufs[b, 0]
            var_f = jnp.exp(2.0 * fls)
            inv_2vp = 0.5 * jnp.exp(-2.0 * pls)
            kl = (pls - fls - 0.5) + (var_f + d * d) * inv_2vp
            acc = acc + jnp.sum(kl.reshape(-1, 8, _C), axis=0)

            @pl.when(ci + _NBUF < _NCHUNK)
            def _():
                issue(b, ci + _NBUF)
        return acc

    acc = lax.fori_loop(0, _NCHUNK // _NBUF, group_body,
                        jnp.zeros((8, _C), jnp.float32))
    out_ref[...] = jnp.sum(acc)[None, None]


def kernel(present_mu, present_log_sigma, future_mu, future_log_sigma):
    pmu = present_mu.reshape(_ROWS, _C)
    pls_ = present_log_sigma.reshape(_ROWS, _C)
    fmu = future_mu.reshape(_ROWS, _C)
    fls = future_log_sigma.reshape(_ROWS, _C)

    hbm_spec = pl.BlockSpec(memory_space=pl.ANY)
    out = pl.pallas_call(
        _kl_kernel,
        in_specs=[hbm_spec, hbm_spec, hbm_spec, hbm_spec],
        out_specs=pl.BlockSpec(memory_space=pltpu.MemorySpace.VMEM),
        out_shape=jax.ShapeDtypeStruct((1, 1), jnp.float32),
        scratch_shapes=[
            pltpu.VMEM((_NBUF, 4, _CH_ROWS, _C), jnp.float32),
            pltpu.SemaphoreType.DMA((_NBUF, 4)),
        ],
    )(pmu, pls_, fmu, fls)
    return out[0, 0] / jnp.float32(_ROWS)


# speedup vs baseline: 1.0244x; 1.0244x over previous
import jax
import jax.numpy as jnp
from jax import lax
from jax.experimental import pallas as pl
from jax.experimental.pallas import tpu as pltpu

_ROWS = 16 * 2048
_C = 256
_CH_ROWS = 1024
_NCHUNK = _ROWS // _CH_ROWS
_NBUF = 8


def _kl_kernel(pmu, pls_, fmu, fls, out_ref, bufs, sems):
    srcs = (pmu, pls_, fmu, fls)

    def dma(b, ci, i):
        r0 = ci * _CH_ROWS
        return pltpu.make_async_copy(
            srcs[i].at[pl.ds(r0, _CH_ROWS), :], bufs.at[b, i], sems.at[b, i])

    def issue(b, ci):
        for i in range(4):
            dma(b, ci, i).start()

    def wait(b, ci):
        for i in range(4):
            dma(b, ci, i).wait()

    for b in range(_NBUF):
        issue(b, b)

    def group_body(g, acc):
        for b in range(_NBUF):
            ci = g * _NBUF + b
            wait(b, ci)
            pls = bufs[b, 1]
            fls = bufs[b, 3]
            d = bufs[b, 2] - bufs[b, 0]
            var_f = jnp.exp(2.0 * fls)
            inv_2vp = 0.5 * jnp.exp(-2.0 * pls)
            kl = (pls - fls - 0.5) + (var_f + d * d) * inv_2vp
            acc = acc + jnp.sum(kl.reshape(-1, 8, _C), axis=0)

            @pl.when(ci + _NBUF < _NCHUNK)
            def _():
                issue(b, ci + _NBUF)
        return acc

    acc = lax.fori_loop(0, _NCHUNK // _NBUF, group_body,
                        jnp.zeros((8, _C), jnp.float32))
    out_ref[...] = jnp.sum(acc)[None, None]


def kernel(present_mu, present_log_sigma, future_mu, future_log_sigma):
    pmu = present_mu.reshape(_ROWS, _C)
    pls_ = present_log_sigma.reshape(_ROWS, _C)
    fmu = future_mu.reshape(_ROWS, _C)
    fls = future_log_sigma.reshape(_ROWS, _C)

    hbm_spec = pl.BlockSpec(memory_space=pl.ANY)
    out = pl.pallas_call(
        _kl_kernel,
        in_specs=[hbm_spec, hbm_spec, hbm_spec, hbm_spec],
        out_specs=pl.BlockSpec(memory_space=pltpu.MemorySpace.VMEM),
        out_shape=jax.ShapeDtypeStruct((1, 1), jnp.float32),
        scratch_shapes=[
            pltpu.VMEM((_NBUF, 4, _CH_ROWS, _C), jnp.float32),
            pltpu.SemaphoreType.DMA((_NBUF, 4)),
        ],
    )(pmu, pls_, fmu, fls)
    return out[0, 0] / jnp.float32(_ROWS)


# speedup vs baseline: 1.0667x; 1.0414x over previous
import jax
import jax.numpy as jnp
from jax import lax
from jax.experimental import pallas as pl
from jax.experimental.pallas import tpu as pltpu

_ROWS = 16 * 2048
_C = 256
_CH_ROWS = 1024
_NCHUNK = _ROWS // _CH_ROWS
_NBUF = 4


def _kl_kernel(pmu, pls_, fmu, fls, out_ref, bufs, sems):
    srcs = (pmu, pls_, fmu, fls)

    def dma(b, ci, i):
        r0 = ci * _CH_ROWS
        return pltpu.make_async_copy(
            srcs[i].at[pl.ds(r0, _CH_ROWS), :], bufs.at[b, i], sems.at[b, i])

    def issue(b, ci):
        for i in range(4):
            dma(b, ci, i).start()

    def wait(b, ci):
        for i in range(4):
            dma(b, ci, i).wait()

    for b in range(_NBUF):
        issue(b, b)

    def group_body(g, acc):
        for b in range(_NBUF):
            ci = g * _NBUF + b
            wait(b, ci)
            pls = bufs[b, 1]
            fls = bufs[b, 3]
            d = bufs[b, 2] - bufs[b, 0]
            var_f = jnp.exp(2.0 * fls)
            inv_2vp = 0.5 * jnp.exp(-2.0 * pls)
            kl = (pls - fls - 0.5) + (var_f + d * d) * inv_2vp
            acc = acc + jnp.sum(kl.reshape(-1, 8, _C), axis=0)

            @pl.when(ci + _NBUF < _NCHUNK)
            def _():
                issue(b, ci + _NBUF)
        return acc

    acc = lax.fori_loop(0, _NCHUNK // _NBUF, group_body,
                        jnp.zeros((8, _C), jnp.float32))
    out_ref[...] = jnp.sum(acc)[None, None]


def kernel(present_mu, present_log_sigma, future_mu, future_log_sigma):
    pmu = present_mu.reshape(_ROWS, _C)
    pls_ = present_log_sigma.reshape(_ROWS, _C)
    fmu = future_mu.reshape(_ROWS, _C)
    fls = future_log_sigma.reshape(_ROWS, _C)

    hbm_spec = pl.BlockSpec(memory_space=pl.ANY)
    out = pl.pallas_call(
        _kl_kernel,
        in_specs=[hbm_spec, hbm_spec, hbm_spec, hbm_spec],
        out_specs=pl.BlockSpec(memory_space=pltpu.MemorySpace.VMEM),
        out_shape=jax.ShapeDtypeStruct((1, 1), jnp.float32),
        scratch_shapes=[
            pltpu.VMEM((_NBUF, 4, _CH_ROWS, _C), jnp.float32),
            pltpu.SemaphoreType.DMA((_NBUF, 4)),
        ],
    )(pmu, pls_, fmu, fls)
    return out[0, 0] / jnp.float32(_ROWS)


# speedup vs baseline: 1.0668x; 1.0000x over previous
import jax
import jax.numpy as jnp
from jax import lax
from jax.experimental import pallas as pl
from jax.experimental.pallas import tpu as pltpu

_ROWS = 16 * 2048
_C = 256
_CH_ROWS = 1024
_NCHUNK = _ROWS // _CH_ROWS
_NBUF = 4


def _kl_kernel(pmu, pls_, fmu, fls, out_ref, bufs, sems):
    srcs = (pmu, pls_, fmu, fls)

    _H = _CH_ROWS // 2

    def dma(b, ci, i, h):
        r0 = ci * _CH_ROWS + h * _H
        return pltpu.make_async_copy(
            srcs[i].at[pl.ds(r0, _H), :],
            bufs.at[b, i, pl.ds(h * _H, _H), :],
            sems.at[b, 2 * i + h])

    def issue(b, ci):
        for i in range(4):
            for h in range(2):
                dma(b, ci, i, h).start()

    def wait(b, ci):
        for i in range(4):
            for h in range(2):
                dma(b, ci, i, h).wait()

    for b in range(_NBUF):
        issue(b, b)

    def group_body(g, acc):
        for b in range(_NBUF):
            ci = g * _NBUF + b
            wait(b, ci)
            pls = bufs[b, 1]
            fls = bufs[b, 3]
            d = bufs[b, 2] - bufs[b, 0]
            var_f = jnp.exp(2.0 * fls)
            inv_2vp = 0.5 * jnp.exp(-2.0 * pls)
            kl = (pls - fls - 0.5) + (var_f + d * d) * inv_2vp
            acc = acc + jnp.sum(kl.reshape(-1, 8, _C), axis=0)

            @pl.when(ci + _NBUF < _NCHUNK)
            def _():
                issue(b, ci + _NBUF)
        return acc

    acc = lax.fori_loop(0, _NCHUNK // _NBUF, group_body,
                        jnp.zeros((8, _C), jnp.float32))
    out_ref[...] = jnp.sum(acc)[None, None]


def kernel(present_mu, present_log_sigma, future_mu, future_log_sigma):
    pmu = present_mu.reshape(_ROWS, _C)
    pls_ = present_log_sigma.reshape(_ROWS, _C)
    fmu = future_mu.reshape(_ROWS, _C)
    fls = future_log_sigma.reshape(_ROWS, _C)

    hbm_spec = pl.BlockSpec(memory_space=pl.ANY)
    out = pl.pallas_call(
        _kl_kernel,
        in_specs=[hbm_spec, hbm_spec, hbm_spec, hbm_spec],
        out_specs=pl.BlockSpec(memory_space=pltpu.MemorySpace.VMEM),
        out_shape=jax.ShapeDtypeStruct((1, 1), jnp.float32),
        scratch_shapes=[
            pltpu.VMEM((_NBUF, 4, _CH_ROWS, _C), jnp.float32),
            pltpu.SemaphoreType.DMA((_NBUF, 8)),
        ],
    )(pmu, pls_, fmu, fls)
    return out[0, 0] / jnp.float32(_ROWS)


# speedup vs baseline: 1.0674x; 1.0006x over previous
import jax
import jax.numpy as jnp
from jax import lax
from jax.experimental import pallas as pl
from jax.experimental.pallas import tpu as pltpu

_ROWS = 16 * 2048
_C = 256
_CH_ROWS = 1024
_NCHUNK = _ROWS // _CH_ROWS
_NBUF = 4


def _kl_kernel(pmu, pls_, fmu, fls, out_ref, bufs, sems):
    srcs = (pmu, pls_, fmu, fls)

    def dma(b, ci, i):
        r0 = ci * _CH_ROWS
        return pltpu.make_async_copy(
            srcs[i].at[pl.ds(r0, _CH_ROWS), :], bufs.at[b, i], sems.at[b, i])

    def issue(b, ci):
        for i in range(4):
            dma(b, ci, i).start()

    def wait(b, ci):
        for i in range(4):
            dma(b, ci, i).wait()

    for b in range(_NBUF):
        issue(b, b)

    def group_body(g, acc):
        for b in range(_NBUF):
            ci = g * _NBUF + b
            wait(b, ci)
            pls = bufs[b, 1]
            fls = bufs[b, 3]
            d = bufs[b, 2] - bufs[b, 0]
            var_f = jnp.exp(2.0 * fls)
            inv_2vp = 0.5 * jnp.exp(-2.0 * pls)
            kl = (pls - fls - 0.5) + (var_f + d * d) * inv_2vp
            acc = acc + jnp.sum(kl.reshape(-1, 8, _C), axis=0)

            @pl.when(ci + _NBUF < _NCHUNK)
            def _():
                issue(b, ci + _NBUF)
        return acc

    acc = lax.fori_loop(0, _NCHUNK // _NBUF, group_body,
                        jnp.zeros((8, _C), jnp.float32))
    out_ref[...] = jnp.sum(acc)[None, None]


def kernel(present_mu, present_log_sigma, future_mu, future_log_sigma):
    pmu = present_mu.reshape(_ROWS, _C)
    pls_ = present_log_sigma.reshape(_ROWS, _C)
    fmu = future_mu.reshape(_ROWS, _C)
    fls = future_log_sigma.reshape(_ROWS, _C)

    hbm_spec = pl.BlockSpec(memory_space=pl.ANY)
    out = pl.pallas_call(
        _kl_kernel,
        in_specs=[hbm_spec, hbm_spec, hbm_spec, hbm_spec],
        out_specs=pl.BlockSpec(memory_space=pltpu.MemorySpace.VMEM),
        out_shape=jax.ShapeDtypeStruct((1, 1), jnp.float32),
        scratch_shapes=[
            pltpu.VMEM((_NBUF, 4, _CH_ROWS, _C), jnp.float32),
            pltpu.SemaphoreType.DMA((_NBUF, 4)),
        ],
    )(pmu, pls_, fmu, fls)
    return out[0, 0] / jnp.float32(_ROWS)
